# Initial kernel scaffold; baseline (speedup 1.0000x reference)
import jax
import jax.numpy as jnp
from jax.experimental import pallas as pl


def kernel(window_bits, conn_state, conn_out, state_memory, output_memory):
    raise NotImplementedError("write your pallas kernel here")


# speedup vs baseline: 2.1231x; 2.1231x over previous
import functools

import jax
import jax.numpy as jnp
from jax import lax
from jax.experimental import pallas as pl
from jax.experimental.pallas import tpu as pltpu
from jax.experimental.pallas import tpu_sc as plsc

_B = 1024
_T_IN = 1024
_N_ST = 2048
_N_OUT = 1024
_HASH = 8192
_NBITS = 13
_NW = 32


def _addr_body(x_ref, connt_ref, out_ref, *, block_n, threshold):
    j = pl.program_id(0)
    x = x_ref[...]
    if threshold:
        xb = (x > 0.5).astype(jnp.bfloat16)
    else:
        xb = x.astype(jnp.bfloat16)
    connt = connt_ref[...]
    t = x.shape[1]
    t_iota = lax.broadcasted_iota(jnp.int32, (t, block_n), 0)
    wlo = jnp.zeros((t, block_n), jnp.int32)
    whi = jnp.zeros((t, block_n), jnp.int32)
    for k in range(7):
        wlo = wlo + jnp.where(t_iota == connt[k, :][None, :], 1 << k, 0)
    for k in range(7, _NBITS):
        whi = whi + jnp.where(t_iota == connt[k, :][None, :], 1 << (k - 7), 0)
    lo = jnp.dot(xb, wlo.astype(jnp.bfloat16), preferred_element_type=jnp.float32)
    hi = jnp.dot(xb, whi.astype(jnp.bfloat16), preferred_element_type=jnp.float32)
    addr = lo.astype(jnp.int32) + hi.astype(jnp.int32) * 128
    n_iota = lax.broadcasted_iota(jnp.int32, addr.shape, 1) + j * block_n
    out_ref[...] = addr + n_iota * _HASH


def _make_addr_call(n_neurons, block_n, t_dim, threshold, x_dtype):
    return pl.pallas_call(
        functools.partial(_addr_body, block_n=block_n, threshold=threshold),
        grid=(n_neurons // block_n,),
        in_specs=[
            pl.BlockSpec((_B, t_dim), lambda j: (0, 0)),
            pl.BlockSpec((24, block_n), lambda j: (0, j)),
        ],
        out_specs=pl.BlockSpec((_B, block_n), lambda j: (0, j)),
        out_shape=jax.ShapeDtypeStruct((_B, n_neurons), jnp.int32),
    )


def _concat_body(win_ref, out_ref):
    out_ref[:, :_T_IN] = win_ref[...].astype(jnp.float32)
    out_ref[:, _T_IN:] = jnp.zeros((_B, _N_ST), jnp.float32)


_concat_call = pl.pallas_call(
    _concat_body,
    out_shape=jax.ShapeDtypeStruct((_B, _T_IN + _N_ST), jnp.float32),
)


@functools.lru_cache(maxsize=None)
def _make_sc_gather(total, chunk):
    per_w = total // _NW
    n_chunks = per_w // chunk
    mesh = plsc.VectorSubcoreMesh(core_axis_name="c", subcore_axis_name="s")

    @functools.partial(
        pl.kernel,
        mesh=mesh,
        out_type=jax.ShapeDtypeStruct((total,), jnp.float32),
        scratch_types=[
            pltpu.VMEM((chunk,), jnp.int32),
            pltpu.VMEM((chunk,), jnp.float32),
            pltpu.SemaphoreType.DMA,
        ],
    )
    def gather_kernel(table_hbm, idx_hbm, out_hbm, idx_v, rows_v, sem):
        c = lax.axis_index("c")
        s = lax.axis_index("s")
        wid = s * 2 + c
        base = wid * per_w

        def body(i, carry):
            off = base + i * chunk
            pltpu.sync_copy(idx_hbm.at[pl.ds(off, chunk)], idx_v)
            pltpu.async_copy(table_hbm.at[idx_v], rows_v, sem).wait()
            pltpu.sync_copy(rows_v, out_hbm.at[pl.ds(off, chunk)])
            return carry

        lax.fori_loop(0, n_chunks, body, 0)

    return gather_kernel


_addr1_call = _make_addr_call(_N_ST, 256, _T_IN, False, jnp.int32)
_addr2_call = _make_addr_call(_N_OUT, 256, _N_ST, True, jnp.float32)


def kernel(window_bits, conn_state, conn_out, state_memory, output_memory):
    state_layer_input = _concat_call(window_bits)

    idx1 = _addr1_call(window_bits, conn_state.T)
    out1_flat = _make_sc_gather(_B * _N_ST, 2048)(
        state_memory.reshape(-1), idx1.reshape(-1))
    state_layer_output = out1_flat.reshape(_B, _N_ST)

    idx2 = _addr2_call(state_layer_output, conn_out.T)
    out2_flat = _make_sc_gather(_B * _N_OUT, 2048)(
        output_memory.reshape(-1), idx2.reshape(-1))
    output_layer_output = out2_flat.reshape(_B, _N_OUT)

    return (state_layer_input, state_layer_output, state_layer_output,
            output_layer_output)


# speedup vs baseline: 5.4997x; 2.5904x over previous
import functools

import jax
import jax.numpy as jnp
from jax import lax
from jax.experimental import pallas as pl
from jax.experimental.pallas import tpu as pltpu
from jax.experimental.pallas import tpu_sc as plsc

_B = 1024
_T_IN = 1024
_N_ST = 2048
_N_OUT = 1024
_HASH = 8192
_NBITS = 13
_NW = 32
_RING = 4


def _addr_body(x_ref, conn_ref, out_ref, *, block_n, threshold, contract_rhs):
    x = x_ref[...]
    if threshold:
        xb = (x > 0.5).astype(jnp.bfloat16)
    else:
        xb = x.astype(jnp.bfloat16)
    conn = conn_ref[...]
    t = x.shape[contract_rhs]
    t_iota = lax.broadcasted_iota(jnp.int32, (block_n, t), 1)
    wlo = jnp.zeros((block_n, t), jnp.int32)
    whi = jnp.zeros((block_n, t), jnp.int32)
    for k in range(7):
        wlo = wlo + jnp.where(t_iota == conn[:, k][:, None], 1 << k, 0)
    for k in range(7, _NBITS):
        whi = whi + jnp.where(t_iota == conn[:, k][:, None], 1 << (k - 7), 0)
    dn = (((1,), (contract_rhs,)), ((), ()))
    lo = lax.dot_general(wlo.astype(jnp.bfloat16), xb, dn,
                         preferred_element_type=jnp.float32)
    hi = lax.dot_general(whi.astype(jnp.bfloat16), xb, dn,
                         preferred_element_type=jnp.float32)
    out_ref[...] = lo.astype(jnp.int32) + hi.astype(jnp.int32) * 128


def _make_addr_call(n_neurons, block_n, x_shape, threshold, contract_rhs):
    return pl.pallas_call(
        functools.partial(_addr_body, block_n=block_n, threshold=threshold,
                          contract_rhs=contract_rhs),
        grid=(n_neurons // block_n,),
        in_specs=[
            pl.BlockSpec(x_shape, lambda j: (0, 0)),
            pl.BlockSpec((block_n, 24), lambda j: (j, 0)),
        ],
        out_specs=pl.BlockSpec((block_n, _B), lambda j: (j, 0)),
        out_shape=jax.ShapeDtypeStruct((n_neurons, _B), jnp.int32),
    )


def _concat_body(win_ref, out_ref):
    out_ref[:, :_T_IN] = win_ref[...].astype(jnp.float32)
    out_ref[:, _T_IN:] = jnp.zeros((_B, _N_ST), jnp.float32)


_concat_call = pl.pallas_call(
    _concat_body,
    out_shape=jax.ShapeDtypeStruct((_B, _T_IN + _N_ST), jnp.float32),
)


def _transpose_body(x_ref, out_ref):
    out_ref[...] = x_ref[...].T


def _make_transpose(n_rows):
    blk = 512
    return pl.pallas_call(
        _transpose_body,
        grid=(n_rows // blk, _B // blk),
        in_specs=[pl.BlockSpec((blk, blk), lambda i, j: (i, j))],
        out_specs=pl.BlockSpec((blk, blk), lambda i, j: (j, i)),
        out_shape=jax.ShapeDtypeStruct((_B, n_rows), jnp.float32),
    )


@functools.lru_cache(maxsize=None)
def _make_sc_gather(n_neurons):
    per_w = n_neurons // _NW
    mesh = plsc.VectorSubcoreMesh(core_axis_name="c", subcore_axis_name="s")

    @functools.partial(
        pl.kernel,
        mesh=mesh,
        out_type=jax.ShapeDtypeStruct((n_neurons, _B), jnp.float32),
        compiler_params=pltpu.CompilerParams(needs_layout_passes=False),
        scratch_types=(
            [pltpu.VMEM((_HASH,), jnp.float32) for _ in range(_RING)]
            + [pltpu.VMEM((_B,), jnp.int32) for _ in range(_RING)]
            + [pltpu.VMEM((_B,), jnp.float32) for _ in range(2)]
            + [
                pltpu.SemaphoreType.DMA,
                pltpu.SemaphoreType.DMA,
                pltpu.SemaphoreType.DMA,
            ]
        ),
    )
    def gather_kernel(table_hbm, addr_hbm, out_hbm, *scratch):
        rows_v = scratch[:_RING]
        idx_v = scratch[_RING:2 * _RING]
        out_v = scratch[2 * _RING:2 * _RING + 2]
        rsem, isem, osem = scratch[2 * _RING + 2:]
        c = lax.axis_index("c")
        s = lax.axis_index("s")
        base = (s * 2 + c) * per_w

        for p in range(_RING - 1):
            pltpu.async_copy(table_hbm.at[base + p], rows_v[p], rsem)
            pltpu.async_copy(addr_hbm.at[base + p], idx_v[p], isem)

        def group(g, carry):
            for sl in range(_RING):
                i = g * _RING + sl
                nf = i + _RING - 1

                @pl.when(nf < per_w)
                def _():
                    pltpu.async_copy(table_hbm.at[base + nf],
                                     rows_v[(sl - 1) % _RING], rsem)
                    pltpu.async_copy(addr_hbm.at[base + nf],
                                     idx_v[(sl - 1) % _RING], isem)

                pltpu.make_async_copy(table_hbm.at[base + i],
                                      rows_v[sl], rsem).wait()
                pltpu.make_async_copy(addr_hbm.at[base + i],
                                      idx_v[sl], isem).wait()

                @pl.when(i >= 2)
                def _():
                    pltpu.make_async_copy(out_v[sl % 2],
                                          out_hbm.at[base + i], osem).wait()

                row = rows_v[sl]
                idx = idx_v[sl]
                ov = out_v[sl % 2]

                def gather16(v, _):
                    for u in range(8):
                        off = v * 128 + u * 16
                        cols = idx[pl.ds(off, 16)]
                        ov[pl.ds(off, 16)] = plsc.load_gather(row, [cols])
                    return _

                lax.fori_loop(0, _B // 128, gather16, 0, unroll=True)
                pltpu.async_copy(ov, out_hbm.at[base + i], osem)
            return carry

        lax.fori_loop(0, per_w // _RING, group, 0)

        pltpu.make_async_copy(out_v[0], out_hbm.at[base], osem).wait()
        pltpu.make_async_copy(out_v[1], out_hbm.at[base], osem).wait()

    return gather_kernel


_addr1_call = _make_addr_call(_N_ST, 256, (_B, _T_IN), False, 1)
_addr2_call = _make_addr_call(_N_OUT, 256, (_N_ST, _B), True, 0)


def kernel(window_bits, conn_state, conn_out, state_memory, output_memory):
    state_layer_input = _concat_call(window_bits)

    addr1t = _addr1_call(window_bits, conn_state)
    out1t = _make_sc_gather(_N_ST)(state_memory, addr1t)

    addr2t = _addr2_call(out1t, conn_out)
    out2t = _make_sc_gather(_N_OUT)(output_memory, addr2t)

    state_layer_output = _make_transpose(_N_ST)(out1t)
    output_layer_output = _make_transpose(_N_OUT)(out2t)

    return (state_layer_input, state_layer_output, state_layer_output,
            output_layer_output)
